# baseline (speedup 1.0000x reference)
import functools

import jax
import jax.numpy as jnp
from jax import lax
from jax.experimental import pallas as pl
from jax.experimental.pallas import tpu as pltpu
from jax.experimental.pallas import tpu_sc as plsc

N_NODES = 10000
N_PAD = 10112
N_GRAPHS = 16

ROW_BLK = 1000
NUM_CORES = 2
NUM_SUBCORES = 16
NW = NUM_CORES * NUM_SUBCORES
CHUNK = 40
NSET = 5
UNROLL = 10
ZROWS = 8


def _proj_l1_tc(x, wk, bk, wq, bq, wv, bv, ws, b):
    n, din = x.shape
    dout = wk.shape[1]
    nblk = n // ROW_BLK

    def body(x_ref, wk_r, bk_r, wq_r, bq_r, wv_r, bv_r, ws_r, b_r,
             ok_r, oq_r, ov_r, os_r):
        xb = x_ref[...]
        ok_r[...] = jnp.dot(xb, wk_r[...], preferred_element_type=jnp.float32) + bk_r[...]
        oq_r[...] = jnp.dot(xb, wq_r[...], preferred_element_type=jnp.float32) + bq_r[...]
        ov_r[...] = jnp.dot(xb, wv_r[...], preferred_element_type=jnp.float32) + bv_r[...]
        os_r[...] = jnp.dot(xb, ws_r[...], preferred_element_type=jnp.float32) + b_r[...]

    wspec = pl.BlockSpec((din, dout), lambda i: (0, 0))
    bspec = pl.BlockSpec((1, dout), lambda i: (0, 0))
    ospec = pl.BlockSpec((ROW_BLK, dout), lambda i: (i, 0))
    out = pl.pallas_call(
        body,
        grid=(nblk,),
        in_specs=[pl.BlockSpec((ROW_BLK, din), lambda i: (i, 0)),
                  wspec, bspec, wspec, bspec, wspec, bspec, wspec, bspec],
        out_specs=[ospec, ospec, ospec, ospec],
        out_shape=[jax.ShapeDtypeStruct((n, dout), jnp.float32)] * 4,
    )(x, wk, bk, wq, bq, wv, bv, ws, b)
    return out


def _combine_proj_l2_tc(parts, skip, wk, bk, wq, bq, wv, bv, ws, b):
    n, din = skip.shape
    dout = wk.shape[1]
    nblk = n // ROW_BLK

    def body(p_ref, s_ref, wk_r, bk_r, wq_r, bq_r, wv_r, bv_r, ws_r, b_r,
             okp_r, oqv_r, os_r):
        h = jnp.maximum(p_ref[0] + p_ref[1] + s_ref[...], 0.0)
        okp_r[:, :dout] = jnp.dot(h, wk_r[...], preferred_element_type=jnp.float32) + bk_r[...]
        okp_r[:, dout:] = jnp.zeros((ROW_BLK, 128 - dout), jnp.float32)
        oqv_r[:, :dout] = jnp.dot(h, wq_r[...], preferred_element_type=jnp.float32) + bq_r[...]
        oqv_r[:, dout:] = jnp.dot(h, wv_r[...], preferred_element_type=jnp.float32) + bv_r[...]
        os_r[...] = jnp.dot(h, ws_r[...], preferred_element_type=jnp.float32) + b_r[...]

    wspec = pl.BlockSpec((din, dout), lambda i: (0, 0))
    bspec = pl.BlockSpec((1, dout), lambda i: (0, 0))
    out = pl.pallas_call(
        body,
        grid=(nblk,),
        in_specs=[pl.BlockSpec((NUM_CORES, ROW_BLK, din), lambda i: (0, i, 0)),
                  pl.BlockSpec((ROW_BLK, din), lambda i: (i, 0)),
                  wspec, bspec, wspec, bspec, wspec, bspec, wspec, bspec],
        out_specs=[pl.BlockSpec((ROW_BLK, 128), lambda i: (i, 0)),
                   pl.BlockSpec((ROW_BLK, 128), lambda i: (i, 0)),
                   pl.BlockSpec((ROW_BLK, dout), lambda i: (i, 0))],
        out_shape=[jax.ShapeDtypeStruct((n, 128), jnp.float32),
                   jax.ShapeDtypeStruct((n, 128), jnp.float32),
                   jax.ShapeDtypeStruct((n, dout), jnp.float32)],
    )(parts, skip, wk, bk, wq, bq, wv, bv, ws, b)
    return out


def _final_tc(parts, skip, batch2d, wf_pad, bf_pad):
    n, dh = skip.shape
    nblk = n // ROW_BLK

    def body(p_ref, s_ref, b_ref, wf_ref, bf_ref, out_ref, sums, cnts):
        i = pl.program_id(0)
        h = jnp.maximum(p_ref[0, :, :dh] + p_ref[1, :, :dh] + s_ref[...], 0.0)
        gids = lax.broadcasted_iota(jnp.int32, (ROW_BLK, N_GRAPHS), 1)
        onehot = (gids == b_ref[...]).astype(jnp.float32)
        psum = lax.dot_general(onehot, h, (((0,), (0,)), ((), ())),
                               preferred_element_type=jnp.float32)
        pcnt = lax.dot_general(onehot, jnp.ones((ROW_BLK, 128), jnp.float32),
                               (((0,), (0,)), ((), ())),
                               preferred_element_type=jnp.float32)

        @pl.when(i == 0)
        def _():
            sums[...] = psum
            cnts[...] = pcnt

        @pl.when(i > 0)
        def _():
            sums[...] = sums[...] + psum
            cnts[...] = cnts[...] + pcnt

        @pl.when(i == nblk - 1)
        def _():
            pooled = sums[...] / jnp.maximum(cnts[...][:, :dh], 1.0)
            out_ref[...] = (jnp.dot(pooled, wf_ref[...],
                                    preferred_element_type=jnp.float32)
                            + bf_ref[...])

    out = pl.pallas_call(
        body,
        grid=(nblk,),
        in_specs=[pl.BlockSpec((NUM_CORES, ROW_BLK, 128), lambda i: (0, i, 0)),
                  pl.BlockSpec((ROW_BLK, dh), lambda i: (i, 0)),
                  pl.BlockSpec((ROW_BLK, 1), lambda i: (i, 0)),
                  pl.BlockSpec((dh, 128), lambda i: (0, 0)),
                  pl.BlockSpec((1, 128), lambda i: (0, 0))],
        out_specs=pl.BlockSpec((N_GRAPHS, 128), lambda i: (0, 0)),
        out_shape=jax.ShapeDtypeStruct((N_GRAPHS, 128), jnp.float32),
        scratch_shapes=[pltpu.VMEM((N_GRAPHS, dh), jnp.float32),
                        pltpu.VMEM((N_GRAPHS, 128), jnp.float32)],
    )(parts, skip, batch2d, wf_pad, bf_pad)
    return out


def _edge_pass_sc(ktab, qtab, vtab, src, dst, h):
    packed = vtab is None
    n = ktab.shape[0]
    e = src.shape[0]
    per_tile = e // NW
    nchunks = per_tile // CHUNK
    nouter = nchunks // UNROLL
    rows_per_tile = N_PAD // NUM_SUBCORES
    nflush = rows_per_tile // ZROWS
    hvec = h // 16
    mesh = plsc.VectorSubcoreMesh(core_axis_name="c", subcore_axis_name="s")

    scratch = [
        pltpu.VMEM((NSET, CHUNK), jnp.int32),
        pltpu.VMEM((NSET, CHUNK), jnp.int32),
        pltpu.VMEM((NSET, CHUNK, 128), jnp.float32),
        pltpu.VMEM((2, CHUNK, 128), jnp.float32),
    ]
    if not packed:
        scratch.append(pltpu.VMEM((2, CHUNK, 128), jnp.float32))
    scratch += [
        pltpu.VMEM((ZROWS, 128), jnp.float32),
        pltpu.VMEM_SHARED((N_PAD, 128), jnp.float32),
        pltpu.SemaphoreType.DMA((NSET,)),
        pltpu.SemaphoreType.DMA((NSET,)),
        pltpu.SemaphoreType.DMA((NSET,)),
    ]

    def body(*refs):
        if packed:
            (k_hbm, q_hbm, src_hbm, dst_hbm, out_hbm,
             si, di, kd, qb, zbuf, agg, isem, gsem, ssem) = refs
            v_hbm = vb = None
        else:
            (k_hbm, q_hbm, v_hbm, src_hbm, dst_hbm, out_hbm,
             si, di, kd, qb, vb, zbuf, agg, isem, gsem, ssem) = refs
        c = lax.axis_index("c")
        s = lax.axis_index("s")
        wid = c * NUM_SUBCORES + s
        row0 = pl.multiple_of(s * rows_per_tile, 8)

        zv = jnp.zeros((16,), jnp.float32)

        def zrow(j, carry):
            for ii in range(8):
                zbuf[j, pl.ds(ii * 16, 16)] = zv
            return carry

        lax.fori_loop(0, ZROWS, zrow, 0)

        def zcopy(t, carry):
            r = pl.multiple_of(row0 + t * ZROWS, 8)
            pltpu.sync_copy(zbuf, agg.at[pl.ds(r, ZROWS)])
            return carry

        lax.fori_loop(0, nflush, zcopy, 0)
        plsc.subcore_barrier()

        base0 = wid * per_tile

        def iload(t, js):
            base = pl.multiple_of(base0 + t * CHUNK, 8)
            pltpu.async_copy(src_hbm.at[pl.ds(base, CHUNK)], si.at[js],
                             isem.at[js])
            pltpu.async_copy(dst_hbm.at[pl.ds(base, CHUNK)], di.at[js],
                             isem.at[js])

        def iwait(t, js):
            base = pl.multiple_of(base0 + t * CHUNK, 8)
            pltpu.make_async_copy(src_hbm.at[pl.ds(base, CHUNK)], si.at[js],
                                  isem.at[js]).wait()
            pltpu.make_async_copy(dst_hbm.at[pl.ds(base, CHUNK)], di.at[js],
                                  isem.at[js]).wait()

        def gstart(jk, jq):
            pltpu.async_copy(k_hbm.at[di.at[jk]], kd.at[jk], gsem.at[jk])
            pltpu.async_copy(q_hbm.at[si.at[jk]], qb.at[jq], gsem.at[jk])
            if not packed:
                pltpu.async_copy(v_hbm.at[si.at[jk]], vb.at[jq], gsem.at[jk])

        def gwait(jk, jq):
            pltpu.make_async_copy(k_hbm.at[di.at[jk]], kd.at[jk],
                                  gsem.at[jk]).wait()
            pltpu.make_async_copy(q_hbm.at[si.at[jk]], qb.at[jq],
                                  gsem.at[jk]).wait()
            if not packed:
                pltpu.make_async_copy(v_hbm.at[si.at[jk]], vb.at[jq],
                                      gsem.at[jk]).wait()

        def sstart(jk):
            pltpu.async_copy(kd.at[jk], agg.at[di.at[jk]], ssem.at[jk],
                             add=True)

        def swait(jk):
            pltpu.make_async_copy(kd.at[jk], agg.at[di.at[jk]],
                                  ssem.at[jk]).wait()

        def compute(jk, jq):
            for r in range(CHUNK):
                for ii in range(hvec):
                    sl = pl.ds(ii * 16, 16)
                    z = kd[jk, r, sl] + qb[jq, r, sl]
                    if packed:
                        v = qb[jq, r, pl.ds(h + ii * 16, 16)]
                    else:
                        v = vb[jq, r, sl]
                    kd[jk, r, sl] = v / (1.0 + jnp.exp(-z))

        iload(0, 0)
        iwait(0, 0)
        gstart(0, 0)
        iload(1, 1)

        def chunk_step(t, carry):
            jk = lax.rem(t, NSET)
            jq = lax.rem(t, 2)
            jk1 = lax.rem(t + 1, NSET)
            jq1 = lax.rem(t + 1, 2)
            jk2 = lax.rem(t + 2, NSET)

            @pl.when(t >= 3)
            def _():
                swait(jk2)

            @pl.when(t + 1 < nchunks)
            def _():
                iwait(t + 1, jk1)
                gstart(jk1, jq1)

            gwait(jk, jq)

            @pl.when(t + 2 < nchunks)
            def _():
                iload(t + 2, jk2)

            compute(jk, jq)
            sstart(jk)
            return carry

        lax.fori_loop(0, nchunks, chunk_step, 0)
        swait((nchunks - 3) % NSET)
        swait((nchunks - 2) % NSET)
        swait((nchunks - 1) % NSET)
        plsc.subcore_barrier()

        def fcopy(t, carry):
            r = pl.multiple_of(row0 + t * ZROWS, 8)
            pltpu.sync_copy(agg.at[pl.ds(r, ZROWS)],
                            out_hbm.at[c, pl.ds(r, ZROWS)])
            return carry

        lax.fori_loop(0, nflush, fcopy, 0)

    ek = pl.kernel(
        body, mesh=mesh,
        out_type=jax.ShapeDtypeStruct((NUM_CORES, N_PAD, 128), jnp.float32),
        scratch_types=scratch)
    if packed:
        return ek(ktab, qtab, src, dst)
    return ek(ktab, qtab, vtab, src, dst)


def kernel(x, Wk1, bk1, Wq1, bq1, Wv1, bv1, Ws1, b1,
           Wk2, bk2, Wq2, bq2, Wv2, bv2, Ws2, b2, Wf, bf,
           edge_index, batch):
    src = edge_index[0]
    dst = edge_index[1]

    b2d = lambda b: b.reshape(1, -1)
    k1, q1, v1, s1 = _proj_l1_tc(x, Wk1, b2d(bk1), Wq1, b2d(bq1),
                                 Wv1, b2d(bv1), Ws1, b2d(b1))
    parts1 = _edge_pass_sc(k1, q1, v1, src, dst, 128)
    k2p, qv2, s2 = _combine_proj_l2_tc(parts1, s1, Wk2, b2d(bk2), Wq2, b2d(bq2),
                                       Wv2, b2d(bv2), Ws2, b2d(b2))
    parts2 = _edge_pass_sc(k2p, qv2, None, src, dst, 64)

    wf_pad = jnp.zeros((Wf.shape[0], 128), jnp.float32).at[:, :Wf.shape[1]].set(Wf)
    bf_pad = jnp.zeros((1, 128), jnp.float32).at[0, :bf.shape[0]].set(bf)
    out_pad = _final_tc(parts2, s2, batch.reshape(-1, 1), wf_pad, bf_pad)
    return out_pad[:, :Wf.shape[1]]


# speedup vs baseline: 4.5763x; 4.5763x over previous
import functools

import jax
import jax.numpy as jnp
from jax import lax
from jax.experimental import pallas as pl
from jax.experimental.pallas import tpu as pltpu
from jax.experimental.pallas import tpu_sc as plsc

N_NODES = 10000
N_PAD = 10112
N_GRAPHS = 16

ROW_BLK = 1000
NUM_CORES = 2
NUM_SUBCORES = 16
NW = NUM_CORES * NUM_SUBCORES
CHUNK = 40
NSET = 5
UNROLL = 10
ZROWS = 8


def _proj_l1_tc(x, wk, bk, wq, bq, wv, bv, ws, b):
    n, din = x.shape
    dout = wk.shape[1]
    nblk = n // ROW_BLK

    def body(x_ref, wk_r, bk_r, wq_r, bq_r, wv_r, bv_r, ws_r, b_r,
             ok_r, oq_r, ov_r, os_r):
        xb = x_ref[...]
        ok_r[...] = jnp.dot(xb, wk_r[...], preferred_element_type=jnp.float32) + bk_r[...]
        oq_r[...] = jnp.dot(xb, wq_r[...], preferred_element_type=jnp.float32) + bq_r[...]
        ov_r[...] = jnp.dot(xb, wv_r[...], preferred_element_type=jnp.float32) + bv_r[...]
        os_r[...] = jnp.dot(xb, ws_r[...], preferred_element_type=jnp.float32) + b_r[...]

    wspec = pl.BlockSpec((din, dout), lambda i: (0, 0))
    bspec = pl.BlockSpec((1, dout), lambda i: (0, 0))
    ospec = pl.BlockSpec((ROW_BLK, dout), lambda i: (i, 0))
    out = pl.pallas_call(
        body,
        grid=(nblk,),
        in_specs=[pl.BlockSpec((ROW_BLK, din), lambda i: (i, 0)),
                  wspec, bspec, wspec, bspec, wspec, bspec, wspec, bspec],
        out_specs=[ospec, ospec, ospec, ospec],
        out_shape=[jax.ShapeDtypeStruct((n, dout), jnp.float32)] * 4,
    )(x, wk, bk, wq, bq, wv, bv, ws, b)
    return out


def _combine_proj_l2_tc(parts, skip, wk, bk, wq, bq, wv, bv, ws, b):
    n, din = skip.shape
    dout = wk.shape[1]
    nblk = n // ROW_BLK

    def body(p_ref, s_ref, wk_r, bk_r, wq_r, bq_r, wv_r, bv_r, ws_r, b_r,
             okp_r, oqv_r, os_r):
        h = jnp.maximum(p_ref[0] + p_ref[1] + s_ref[...], 0.0)
        okp_r[:, :dout] = jnp.dot(h, wk_r[...], preferred_element_type=jnp.float32) + bk_r[...]
        okp_r[:, dout:] = jnp.zeros((ROW_BLK, 128 - dout), jnp.float32)
        oqv_r[:, :dout] = jnp.dot(h, wq_r[...], preferred_element_type=jnp.float32) + bq_r[...]
        oqv_r[:, dout:] = jnp.dot(h, wv_r[...], preferred_element_type=jnp.float32) + bv_r[...]
        os_r[...] = jnp.dot(h, ws_r[...], preferred_element_type=jnp.float32) + b_r[...]

    wspec = pl.BlockSpec((din, dout), lambda i: (0, 0))
    bspec = pl.BlockSpec((1, dout), lambda i: (0, 0))
    out = pl.pallas_call(
        body,
        grid=(nblk,),
        in_specs=[pl.BlockSpec((NUM_CORES, ROW_BLK, din), lambda i: (0, i, 0)),
                  pl.BlockSpec((ROW_BLK, din), lambda i: (i, 0)),
                  wspec, bspec, wspec, bspec, wspec, bspec, wspec, bspec],
        out_specs=[pl.BlockSpec((ROW_BLK, 128), lambda i: (i, 0)),
                   pl.BlockSpec((ROW_BLK, 128), lambda i: (i, 0)),
                   pl.BlockSpec((ROW_BLK, dout), lambda i: (i, 0))],
        out_shape=[jax.ShapeDtypeStruct((n, 128), jnp.float32),
                   jax.ShapeDtypeStruct((n, 128), jnp.float32),
                   jax.ShapeDtypeStruct((n, dout), jnp.float32)],
    )(parts, skip, wk, bk, wq, bq, wv, bv, ws, b)
    return out


def _final_tc(parts, skip, batch2d, wf_pad, bf_pad):
    n, dh = skip.shape
    nblk = n // ROW_BLK

    def body(p_ref, s_ref, b_ref, wf_ref, bf_ref, out_ref, sums, cnts):
        i = pl.program_id(0)
        h = jnp.maximum(p_ref[0, :, :dh] + p_ref[1, :, :dh] + s_ref[...], 0.0)
        gids = lax.broadcasted_iota(jnp.int32, (ROW_BLK, N_GRAPHS), 1)
        onehot = (gids == b_ref[...]).astype(jnp.float32)
        psum = lax.dot_general(onehot, h, (((0,), (0,)), ((), ())),
                               preferred_element_type=jnp.float32)
        pcnt = lax.dot_general(onehot, jnp.ones((ROW_BLK, 128), jnp.float32),
                               (((0,), (0,)), ((), ())),
                               preferred_element_type=jnp.float32)

        @pl.when(i == 0)
        def _():
            sums[...] = psum
            cnts[...] = pcnt

        @pl.when(i > 0)
        def _():
            sums[...] = sums[...] + psum
            cnts[...] = cnts[...] + pcnt

        @pl.when(i == nblk - 1)
        def _():
            pooled = sums[...] / jnp.maximum(cnts[...][:, :dh], 1.0)
            out_ref[...] = (jnp.dot(pooled, wf_ref[...],
                                    preferred_element_type=jnp.float32)
                            + bf_ref[...])

    out = pl.pallas_call(
        body,
        grid=(nblk,),
        in_specs=[pl.BlockSpec((NUM_CORES, ROW_BLK, 128), lambda i: (0, i, 0)),
                  pl.BlockSpec((ROW_BLK, dh), lambda i: (i, 0)),
                  pl.BlockSpec((ROW_BLK, 1), lambda i: (i, 0)),
                  pl.BlockSpec((dh, 128), lambda i: (0, 0)),
                  pl.BlockSpec((1, 128), lambda i: (0, 0))],
        out_specs=pl.BlockSpec((N_GRAPHS, 128), lambda i: (0, 0)),
        out_shape=jax.ShapeDtypeStruct((N_GRAPHS, 128), jnp.float32),
        scratch_shapes=[pltpu.VMEM((N_GRAPHS, dh), jnp.float32),
                        pltpu.VMEM((N_GRAPHS, 128), jnp.float32)],
    )(parts, skip, batch2d, wf_pad, bf_pad)
    return out


def _edge_pass_sc(ktab, qtab, vtab, src, dst, h):
    packed = vtab is None
    n = ktab.shape[0]
    e = src.shape[0]
    per_tile = e // NW
    nchunks = per_tile // CHUNK
    nouter = nchunks // UNROLL
    rows_per_tile = N_PAD // NUM_SUBCORES
    nflush = rows_per_tile // ZROWS
    hvec = h // 16
    mesh = plsc.VectorSubcoreMesh(core_axis_name="c", subcore_axis_name="s")

    scratch = [
        pltpu.VMEM((NSET, CHUNK), jnp.int32),
        pltpu.VMEM((NSET, CHUNK), jnp.int32),
        pltpu.VMEM((NSET, CHUNK, 128), jnp.float32),
        pltpu.VMEM((2, CHUNK, 128), jnp.float32),
    ]
    if not packed:
        scratch.append(pltpu.VMEM((2, CHUNK, 128), jnp.float32))
    scratch += [
        pltpu.VMEM((ZROWS, 128), jnp.float32),
        pltpu.VMEM_SHARED((N_PAD, 128), jnp.float32),
        pltpu.SemaphoreType.DMA((NSET,)),
        pltpu.SemaphoreType.DMA((NSET,)),
        pltpu.SemaphoreType.DMA((NSET,)),
    ]

    def body(*refs):
        if packed:
            (k_hbm, q_hbm, src_hbm, dst_hbm, out_hbm,
             si, di, kd, qb, zbuf, agg, isem, gsem, ssem) = refs
            v_hbm = vb = None
        else:
            (k_hbm, q_hbm, v_hbm, src_hbm, dst_hbm, out_hbm,
             si, di, kd, qb, vb, zbuf, agg, isem, gsem, ssem) = refs
        c = lax.axis_index("c")
        s = lax.axis_index("s")
        wid = c * NUM_SUBCORES + s
        row0 = pl.multiple_of(s * rows_per_tile, 8)

        zv = jnp.zeros((16,), jnp.float32)

        def zrow(j, carry):
            for ii in range(8):
                zbuf[j, pl.ds(ii * 16, 16)] = zv
            return carry

        lax.fori_loop(0, ZROWS, zrow, 0)

        def zcopy(t, carry):
            r = pl.multiple_of(row0 + t * ZROWS, 8)
            pltpu.sync_copy(zbuf, agg.at[pl.ds(r, ZROWS)])
            return carry

        lax.fori_loop(0, nflush, zcopy, 0)
        plsc.subcore_barrier()

        base0 = wid * per_tile

        def iload(t, js):
            base = pl.multiple_of(base0 + t * CHUNK, 8)
            pltpu.async_copy(src_hbm.at[pl.ds(base, CHUNK)], si.at[js],
                             isem.at[js])
            pltpu.async_copy(dst_hbm.at[pl.ds(base, CHUNK)], di.at[js],
                             isem.at[js])

        def iwait(t, js):
            base = pl.multiple_of(base0 + t * CHUNK, 8)
            pltpu.make_async_copy(src_hbm.at[pl.ds(base, CHUNK)], si.at[js],
                                  isem.at[js]).wait()
            pltpu.make_async_copy(dst_hbm.at[pl.ds(base, CHUNK)], di.at[js],
                                  isem.at[js]).wait()

        def gstart(jk, jq):
            pltpu.async_copy(k_hbm.at[di.at[jk]], kd.at[jk], gsem.at[jk])
            pltpu.async_copy(q_hbm.at[si.at[jk]], qb.at[jq], gsem.at[jk])
            if not packed:
                pltpu.async_copy(v_hbm.at[si.at[jk]], vb.at[jq], gsem.at[jk])

        def gwait(jk, jq):
            pltpu.make_async_copy(k_hbm.at[di.at[jk]], kd.at[jk],
                                  gsem.at[jk]).wait()
            pltpu.make_async_copy(q_hbm.at[si.at[jk]], qb.at[jq],
                                  gsem.at[jk]).wait()
            if not packed:
                pltpu.make_async_copy(v_hbm.at[si.at[jk]], vb.at[jq],
                                      gsem.at[jk]).wait()

        def sstart(jk):
            pltpu.async_copy(kd.at[jk], agg.at[di.at[jk]], ssem.at[jk],
                             add=True)

        def swait(jk):
            pltpu.make_async_copy(kd.at[jk], agg.at[di.at[jk]],
                                  ssem.at[jk]).wait()

        def compute(jk, jq):
            def crow(r, inner):
                for ii in range(hvec):
                    sl = pl.ds(ii * 16, 16)
                    z = kd[jk, r, sl] + qb[jq, r, sl]
                    if packed:
                        v = qb[jq, r, pl.ds(h + ii * 16, 16)]
                    else:
                        v = vb[jq, r, sl]
                    kd[jk, r, sl] = v / (1.0 + jnp.exp(-z))
                return inner

            lax.fori_loop(0, CHUNK, crow, 0)

        iload(0, 0)
        iwait(0, 0)
        gstart(0, 0)
        iload(1, 1)

        def outer(g, carry):
            for j in range(UNROLL):
                t = g * UNROLL + j
                jk, jq = j % NSET, j % 2

                if j >= 3:
                    swait((j - 3) % NSET)
                else:
                    @pl.when(g > 0)
                    def _(jj=(j - 3) % NSET):
                        swait(jj)

                if j < UNROLL - 1:
                    iwait(t + 1, (j + 1) % NSET)
                    gstart((j + 1) % NSET, (j + 1) % 2)
                else:
                    @pl.when(g < nouter - 1)
                    def _():
                        iwait(t + 1, (j + 1) % NSET)
                        gstart((j + 1) % NSET, (j + 1) % 2)

                gwait(jk, jq)

                if j < UNROLL - 2:
                    iload(t + 2, (j + 2) % NSET)
                else:
                    @pl.when(g < nouter - 1)
                    def _():
                        iload(t + 2, (j + 2) % NSET)

                compute(jk, jq)
                sstart(jk)
            return carry

        lax.fori_loop(0, nouter, outer, 0)
        swait((nchunks - 3) % NSET)
        swait((nchunks - 2) % NSET)
        swait((nchunks - 1) % NSET)
        plsc.subcore_barrier()

        def fcopy(t, carry):
            r = pl.multiple_of(row0 + t * ZROWS, 8)
            pltpu.sync_copy(agg.at[pl.ds(r, ZROWS)],
                            out_hbm.at[c, pl.ds(r, ZROWS)])
            return carry

        lax.fori_loop(0, nflush, fcopy, 0)

    ek = pl.kernel(
        body, mesh=mesh,
        out_type=jax.ShapeDtypeStruct((NUM_CORES, N_PAD, 128), jnp.float32),
        scratch_types=scratch)
    if packed:
        return ek(ktab, qtab, src, dst)
    return ek(ktab, qtab, vtab, src, dst)


def kernel(x, Wk1, bk1, Wq1, bq1, Wv1, bv1, Ws1, b1,
           Wk2, bk2, Wq2, bq2, Wv2, bv2, Ws2, b2, Wf, bf,
           edge_index, batch):
    src = edge_index[0]
    dst = edge_index[1]

    b2d = lambda b: b.reshape(1, -1)
    k1, q1, v1, s1 = _proj_l1_tc(x, Wk1, b2d(bk1), Wq1, b2d(bq1),
                                 Wv1, b2d(bv1), Ws1, b2d(b1))
    parts1 = _edge_pass_sc(k1, q1, v1, src, dst, 128)
    k2p, qv2, s2 = _combine_proj_l2_tc(parts1, s1, Wk2, b2d(bk2), Wq2, b2d(bq2),
                                       Wv2, b2d(bv2), Ws2, b2d(b2))
    parts2 = _edge_pass_sc(k2p, qv2, None, src, dst, 64)

    wf_pad = jnp.zeros((Wf.shape[0], 128), jnp.float32).at[:, :Wf.shape[1]].set(Wf)
    bf_pad = jnp.zeros((1, 128), jnp.float32).at[0, :bf.shape[0]].set(bf)
    out_pad = _final_tc(parts2, s2, batch.reshape(-1, 1), wf_pad, bf_pad)
    return out_pad[:, :Wf.shape[1]]


# speedup vs baseline: 5.0739x; 1.1087x over previous
import functools

import jax
import jax.numpy as jnp
from jax import lax
from jax.experimental import pallas as pl
from jax.experimental.pallas import tpu as pltpu
from jax.experimental.pallas import tpu_sc as plsc

N_NODES = 10000
N_PAD = 10112
N_GRAPHS = 16

ROW_BLK = 1000
NUM_CORES = 2
NUM_SUBCORES = 16
NW = NUM_CORES * NUM_SUBCORES
CHUNK = 40
NSET = 5
UNROLL = 10
ZROWS = 8


def _proj_l1_tc(x, wk, bk, wq, bq, wv, bv, ws, b):
    n, din = x.shape
    dout = wk.shape[1]
    nblk = n // ROW_BLK

    def body(x_ref, wk_r, bk_r, wq_r, bq_r, wv_r, bv_r, ws_r, b_r,
             ok_r, oq_r, ov_r, os_r):
        xb = x_ref[...]
        ok_r[...] = jnp.dot(xb, wk_r[...], preferred_element_type=jnp.float32) + bk_r[...]
        oq_r[...] = jnp.dot(xb, wq_r[...], preferred_element_type=jnp.float32) + bq_r[...]
        ov_r[...] = jnp.dot(xb, wv_r[...], preferred_element_type=jnp.float32) + bv_r[...]
        os_r[...] = jnp.dot(xb, ws_r[...], preferred_element_type=jnp.float32) + b_r[...]

    wspec = pl.BlockSpec((din, dout), lambda i: (0, 0))
    bspec = pl.BlockSpec((1, dout), lambda i: (0, 0))
    ospec = pl.BlockSpec((ROW_BLK, dout), lambda i: (i, 0))
    out = pl.pallas_call(
        body,
        grid=(nblk,),
        in_specs=[pl.BlockSpec((ROW_BLK, din), lambda i: (i, 0)),
                  wspec, bspec, wspec, bspec, wspec, bspec, wspec, bspec],
        out_specs=[ospec, ospec, ospec, ospec],
        out_shape=[jax.ShapeDtypeStruct((n, dout), jnp.float32)] * 4,
    )(x, wk, bk, wq, bq, wv, bv, ws, b)
    return out


def _combine_proj_l2_tc(parts, skip, wk, bk, wq, bq, wv, bv, ws, b):
    n, din = skip.shape
    dout = wk.shape[1]
    nblk = n // ROW_BLK

    def body(p_ref, s_ref, wk_r, bk_r, wq_r, bq_r, wv_r, bv_r, ws_r, b_r,
             okp_r, oqv_r, os_r):
        h = jnp.maximum(p_ref[0] + p_ref[1] + s_ref[...], 0.0)
        okp_r[:, :dout] = jnp.dot(h, wk_r[...], preferred_element_type=jnp.float32) + bk_r[...]
        okp_r[:, dout:] = jnp.zeros((ROW_BLK, 128 - dout), jnp.float32)
        oqv_r[:, :dout] = jnp.dot(h, wq_r[...], preferred_element_type=jnp.float32) + bq_r[...]
        oqv_r[:, dout:] = jnp.dot(h, wv_r[...], preferred_element_type=jnp.float32) + bv_r[...]
        os_r[...] = jnp.dot(h, ws_r[...], preferred_element_type=jnp.float32) + b_r[...]

    wspec = pl.BlockSpec((din, dout), lambda i: (0, 0))
    bspec = pl.BlockSpec((1, dout), lambda i: (0, 0))
    out = pl.pallas_call(
        body,
        grid=(nblk,),
        in_specs=[pl.BlockSpec((NUM_CORES, ROW_BLK, din), lambda i: (0, i, 0)),
                  pl.BlockSpec((ROW_BLK, din), lambda i: (i, 0)),
                  wspec, bspec, wspec, bspec, wspec, bspec, wspec, bspec],
        out_specs=[pl.BlockSpec((ROW_BLK, 128), lambda i: (i, 0)),
                   pl.BlockSpec((ROW_BLK, 128), lambda i: (i, 0)),
                   pl.BlockSpec((ROW_BLK, dout), lambda i: (i, 0))],
        out_shape=[jax.ShapeDtypeStruct((n, 128), jnp.float32),
                   jax.ShapeDtypeStruct((n, 128), jnp.float32),
                   jax.ShapeDtypeStruct((n, dout), jnp.float32)],
    )(parts, skip, wk, bk, wq, bq, wv, bv, ws, b)
    return out


def _final_tc(parts, skip, batch2d, wf_pad, bf_pad):
    n, dh = skip.shape
    nblk = n // ROW_BLK

    def body(p_ref, s_ref, b_ref, wf_ref, bf_ref, out_ref, sums, cnts):
        i = pl.program_id(0)
        h = jnp.maximum(p_ref[0, :, :dh] + p_ref[1, :, :dh] + s_ref[...], 0.0)
        gids = lax.broadcasted_iota(jnp.int32, (ROW_BLK, N_GRAPHS), 1)
        onehot = (gids == b_ref[...]).astype(jnp.float32)
        psum = lax.dot_general(onehot, h, (((0,), (0,)), ((), ())),
                               preferred_element_type=jnp.float32)
        pcnt = lax.dot_general(onehot, jnp.ones((ROW_BLK, 128), jnp.float32),
                               (((0,), (0,)), ((), ())),
                               preferred_element_type=jnp.float32)

        @pl.when(i == 0)
        def _():
            sums[...] = psum
            cnts[...] = pcnt

        @pl.when(i > 0)
        def _():
            sums[...] = sums[...] + psum
            cnts[...] = cnts[...] + pcnt

        @pl.when(i == nblk - 1)
        def _():
            pooled = sums[...] / jnp.maximum(cnts[...][:, :dh], 1.0)
            out_ref[...] = (jnp.dot(pooled, wf_ref[...],
                                    preferred_element_type=jnp.float32)
                            + bf_ref[...])

    out = pl.pallas_call(
        body,
        grid=(nblk,),
        in_specs=[pl.BlockSpec((NUM_CORES, ROW_BLK, 128), lambda i: (0, i, 0)),
                  pl.BlockSpec((ROW_BLK, dh), lambda i: (i, 0)),
                  pl.BlockSpec((ROW_BLK, 1), lambda i: (i, 0)),
                  pl.BlockSpec((dh, 128), lambda i: (0, 0)),
                  pl.BlockSpec((1, 128), lambda i: (0, 0))],
        out_specs=pl.BlockSpec((N_GRAPHS, 128), lambda i: (0, 0)),
        out_shape=jax.ShapeDtypeStruct((N_GRAPHS, 128), jnp.float32),
        scratch_shapes=[pltpu.VMEM((N_GRAPHS, dh), jnp.float32),
                        pltpu.VMEM((N_GRAPHS, 128), jnp.float32)],
    )(parts, skip, batch2d, wf_pad, bf_pad)
    return out


def _edge_pass_sc(ktab, qtab, vtab, src, dst, h):
    packed = vtab is None
    n = ktab.shape[0]
    e = src.shape[0]
    per_tile = e // NW
    nchunks = per_tile // CHUNK
    nouter = nchunks // UNROLL
    rows_per_tile = N_PAD // NUM_SUBCORES
    nflush = rows_per_tile // ZROWS
    hvec = h // 16
    mesh = plsc.VectorSubcoreMesh(core_axis_name="c", subcore_axis_name="s")

    scratch = [
        pltpu.VMEM((NSET, CHUNK), jnp.int32),
        pltpu.VMEM((NSET, CHUNK), jnp.int32),
        pltpu.VMEM((NSET, CHUNK, 128), jnp.float32),
        pltpu.VMEM((2, CHUNK, 128), jnp.float32),
    ]
    if not packed:
        scratch.append(pltpu.VMEM((2, CHUNK, 128), jnp.float32))
    scratch += [
        pltpu.VMEM((ZROWS, 128), jnp.float32),
        pltpu.VMEM_SHARED((N_PAD, 128), jnp.float32),
        pltpu.SemaphoreType.DMA((NSET,)),
        pltpu.SemaphoreType.DMA((NSET,)),
        pltpu.SemaphoreType.DMA((NSET,)),
    ]

    def body(*refs):
        if packed:
            (k_hbm, q_hbm, src_hbm, dst_hbm, out_hbm,
             si, di, kd, qb, zbuf, agg, isem, gsem, ssem) = refs
            v_hbm = vb = None
        else:
            (k_hbm, q_hbm, v_hbm, src_hbm, dst_hbm, out_hbm,
             si, di, kd, qb, vb, zbuf, agg, isem, gsem, ssem) = refs
        c = lax.axis_index("c")
        s = lax.axis_index("s")
        wid = c * NUM_SUBCORES + s
        row0 = pl.multiple_of(s * rows_per_tile, 8)

        zv = jnp.zeros((16,), jnp.float32)

        def zrow(j, carry):
            for ii in range(8):
                zbuf[j, pl.ds(ii * 16, 16)] = zv
            return carry

        lax.fori_loop(0, ZROWS, zrow, 0)

        def zcopy(t, carry):
            r = pl.multiple_of(row0 + t * ZROWS, 8)
            pltpu.sync_copy(zbuf, agg.at[pl.ds(r, ZROWS)])
            return carry

        lax.fori_loop(0, nflush, zcopy, 0)
        plsc.subcore_barrier()

        base0 = wid * per_tile

        def iload(t, js):
            base = pl.multiple_of(base0 + t * CHUNK, 8)
            pltpu.async_copy(src_hbm.at[pl.ds(base, CHUNK)], si.at[js],
                             isem.at[js])
            pltpu.async_copy(dst_hbm.at[pl.ds(base, CHUNK)], di.at[js],
                             isem.at[js])

        def iwait(t, js):
            base = pl.multiple_of(base0 + t * CHUNK, 8)
            pltpu.make_async_copy(src_hbm.at[pl.ds(base, CHUNK)], si.at[js],
                                  isem.at[js]).wait()
            pltpu.make_async_copy(dst_hbm.at[pl.ds(base, CHUNK)], di.at[js],
                                  isem.at[js]).wait()

        def gstart(jk, jq):
            pltpu.async_copy(k_hbm.at[di.at[jk]], kd.at[jk], gsem.at[jk])
            pltpu.async_copy(q_hbm.at[si.at[jk]], qb.at[jq], gsem.at[jk])
            if not packed:
                pltpu.async_copy(v_hbm.at[si.at[jk]], vb.at[jq], gsem.at[jk])

        def gwait(jk, jq):
            pltpu.make_async_copy(k_hbm.at[di.at[jk]], kd.at[jk],
                                  gsem.at[jk]).wait()
            pltpu.make_async_copy(q_hbm.at[si.at[jk]], qb.at[jq],
                                  gsem.at[jk]).wait()
            if not packed:
                pltpu.make_async_copy(v_hbm.at[si.at[jk]], vb.at[jq],
                                      gsem.at[jk]).wait()

        def sstart(jk):
            pltpu.async_copy(kd.at[jk], agg.at[di.at[jk]], ssem.at[jk],
                             add=True)

        def swait(jk):
            pltpu.make_async_copy(kd.at[jk], agg.at[di.at[jk]],
                                  ssem.at[jk]).wait()

        def compute(jk, jq):
            def crow(rr, inner):
                r0 = rr * 2
                for u in range(2):
                    r = r0 + u
                    for ii in range(hvec):
                        sl = pl.ds(ii * 16, 16)
                        z = kd[jk, r, sl] + qb[jq, r, sl]
                        if packed:
                            v = qb[jq, r, pl.ds(h + ii * 16, 16)]
                        else:
                            v = vb[jq, r, sl]
                        kd[jk, r, sl] = v / (1.0 + jnp.exp(-z))
                return inner

            lax.fori_loop(0, CHUNK // 2, crow, 0)

        iload(0, 0)
        iwait(0, 0)
        gstart(0, 0)
        iload(1, 1)

        def outer(g, carry):
            for j in range(UNROLL):
                t = g * UNROLL + j
                jk, jq = j % NSET, j % 2

                if j >= 3:
                    swait((j - 3) % NSET)
                else:
                    @pl.when(g > 0)
                    def _(jj=(j - 3) % NSET):
                        swait(jj)

                if j < UNROLL - 1:
                    iwait(t + 1, (j + 1) % NSET)
                    gstart((j + 1) % NSET, (j + 1) % 2)
                else:
                    @pl.when(g < nouter - 1)
                    def _():
                        iwait(t + 1, (j + 1) % NSET)
                        gstart((j + 1) % NSET, (j + 1) % 2)

                gwait(jk, jq)

                if j < UNROLL - 2:
                    iload(t + 2, (j + 2) % NSET)
                else:
                    @pl.when(g < nouter - 1)
                    def _():
                        iload(t + 2, (j + 2) % NSET)

                compute(jk, jq)
                sstart(jk)
            return carry

        lax.fori_loop(0, nouter, outer, 0)
        swait((nchunks - 3) % NSET)
        swait((nchunks - 2) % NSET)
        swait((nchunks - 1) % NSET)
        plsc.subcore_barrier()

        def fcopy(t, carry):
            r = pl.multiple_of(row0 + t * ZROWS, 8)
            pltpu.sync_copy(agg.at[pl.ds(r, ZROWS)],
                            out_hbm.at[c, pl.ds(r, ZROWS)])
            return carry

        lax.fori_loop(0, nflush, fcopy, 0)

    ek = pl.kernel(
        body, mesh=mesh,
        out_type=jax.ShapeDtypeStruct((NUM_CORES, N_PAD, 128), jnp.float32),
        scratch_types=scratch)
    if packed:
        return ek(ktab, qtab, src, dst)
    return ek(ktab, qtab, vtab, src, dst)


def kernel(x, Wk1, bk1, Wq1, bq1, Wv1, bv1, Ws1, b1,
           Wk2, bk2, Wq2, bq2, Wv2, bv2, Ws2, b2, Wf, bf,
           edge_index, batch):
    src = edge_index[0]
    dst = edge_index[1]

    b2d = lambda b: b.reshape(1, -1)
    k1, q1, v1, s1 = _proj_l1_tc(x, Wk1, b2d(bk1), Wq1, b2d(bq1),
                                 Wv1, b2d(bv1), Ws1, b2d(b1))
    parts1 = _edge_pass_sc(k1, q1, v1, src, dst, 128)
    k2p, qv2, s2 = _combine_proj_l2_tc(parts1, s1, Wk2, b2d(bk2), Wq2, b2d(bq2),
                                       Wv2, b2d(bv2), Ws2, b2d(b2))
    parts2 = _edge_pass_sc(k2p, qv2, None, src, dst, 64)

    wf_pad = jnp.zeros((Wf.shape[0], 128), jnp.float32).at[:, :Wf.shape[1]].set(Wf)
    bf_pad = jnp.zeros((1, 128), jnp.float32).at[0, :bf.shape[0]].set(bf)
    out_pad = _final_tc(parts2, s2, batch.reshape(-1, 1), wf_pad, bf_pad)
    return out_pad[:, :Wf.shape[1]]
